# baseline (speedup 1.0000x reference)
import jax
import jax.numpy as jnp
from jax import lax
from jax.experimental import pallas as pl
from jax.experimental.pallas import tpu as pltpu
from jax.experimental.pallas import tpu_sc as plsc

B = 1024
L = 50
G = 10
D = 64
NC = 2
NS = 16
NW = NC * NS
BPW = B // NW
TPW = BPW * L
NIDX = TPW * G
IDX_PER_B = L * G
IDX_PAD = 512
LANES = 16


def _body(sent_ref, wn_ref, wl_ref, out_ref,
          sent_v, idx1_v, ng_v, idx_v, rows_v, out_v, sem0, sems):
    wid = lax.axis_index("s") * NC + lax.axis_index("c")

    pltpu.sync_copy(sent_ref.at[pl.ds(wid * TPW, TPW)], sent_v)

    iota = lax.iota(jnp.int32, LANES)

    def idx1_body(k, _):
        s = k * LANES + iota
        t = lax.shift_right_logical(s * 6554, 16)
        g = s - t * G
        tok = plsc.load_gather(sent_v, [t])
        idx1_v[k // 8, pl.ds((k % 8) * LANES, LANES)] = tok * G + g
        return 0

    lax.fori_loop(0, NIDX // LANES, idx1_body, 0)

    hop1 = [
        pltpu.async_copy(wn_ref.at[idx1_v.at[j]], ng_v.at[j], sem0)
        for j in range(NIDX // 128)
    ]
    for cp in hop1:
        cp.wait()

    def conv_body(t, _):
        b = t // 32
        k = t - b * 32
        off = k * LANES + iota
        valid = off < IDX_PER_B
        p = b * IDX_PER_B + off
        r = jnp.minimum(lax.shift_right_logical(p, 7), NIDX // 128 - 1)
        c = lax.bitwise_and(p, 127)
        v = plsc.load_gather(ng_v, [r, c])
        vi = jnp.where(valid, v.astype(jnp.int32), 0)
        idx_v[b, k // 8, pl.ds((k % 8) * LANES, LANES)] = vi
        return 0

    lax.fori_loop(0, BPW * 32, conv_body, 0)

    def fire(b, par):
        return [
            pltpu.async_copy(wl_ref.at[idx_v.at[b, j]],
                             rows_v.at[par, pl.ds(j * 128, 128)],
                             sems.at[par])
            for j in range(4)
        ]

    pending = {0: fire(0, 0)}
    for b in range(BPW):
        par = b % 2
        if b + 1 < BPW:
            pending[1 - par] = fire(b + 1, 1 - par)
        for cp in pending[par]:
            cp.wait()

        def red_body(rr, accs):
            a0, a1, a2, a3 = accs
            for u in range(4):
                r = rr * 4 + u
                a0 = a0 + rows_v[par, r, pl.ds(0, LANES)]
                a1 = a1 + rows_v[par, r, pl.ds(LANES, LANES)]
                a2 = a2 + rows_v[par, r, pl.ds(2 * LANES, LANES)]
                a3 = a3 + rows_v[par, r, pl.ds(3 * LANES, LANES)]
            return a0, a1, a2, a3

        z = jnp.zeros((LANES,), jnp.float32)
        acc = lax.fori_loop(0, IDX_PAD // 4, red_body, (z, z, z, z))
        for d in range(4):
            out_v[b, pl.ds(d * LANES, LANES)] = acc[d]

    pltpu.sync_copy(out_v, out_ref.at[pl.ds(wid * BPW, BPW)])


@jax.jit
def _run(sent_f, wn_f, wl):
    mesh = plsc.VectorSubcoreMesh(core_axis_name="c", subcore_axis_name="s")
    return pl.kernel(
        _body,
        out_type=jax.ShapeDtypeStruct((B, D), jnp.float32),
        mesh=mesh,
        scratch_types=[
            pltpu.VMEM((TPW,), jnp.int32),
            pltpu.VMEM((NIDX // 128, 128), jnp.int32),
            pltpu.VMEM((NIDX // 128, 128), jnp.float32),
            pltpu.VMEM((BPW, 4, 128), jnp.int32),
            pltpu.VMEM((2, IDX_PAD, D), jnp.float32),
            pltpu.VMEM((BPW, D), jnp.float32),
            pltpu.SemaphoreType.DMA,
            pltpu.SemaphoreType.DMA((2,)),
        ],
        compiler_params=pltpu.CompilerParams(use_tc_tiling_on_sc=False,
                                             needs_layout_passes=False),
    )(sent_f, wn_f, wl)


def kernel(sent, W_ngram_idx, W_lookup):
    sent_f = sent.astype(jnp.int32).reshape(-1)
    wn_f = W_ngram_idx.reshape(-1)
    return _run(sent_f, wn_f, W_lookup)
